# baseline (speedup 1.0000x reference)
import functools

import jax
import jax.numpy as jnp
from jax import lax
from jax.experimental import pallas as pl
from jax.experimental.pallas import tpu as pltpu
from jax.experimental.pallas import tpu_sc as plsc

E = 8
K = 2
H = 768
I = 1536
I2 = 2 * I
T = 2048
P = 2 * T
B = 640
RB = 512
NB = 13
NBE = T // B
PPAD = NB * B
NC = 2
NS = 16
NW = NC * NS


def _router_body(lg_ref, wgt_ref, dest_ref, be_ref, tot_ref, o_scr, cum_scr):
    lg = lg_ref[...].T
    iota_e = lax.broadcasted_iota(jnp.int32, (E, T), 0)
    m1 = jnp.max(lg, axis=0, keepdims=True)
    i1 = jnp.min(jnp.where(lg == m1, iota_e, E), axis=0, keepdims=True)
    masked = jnp.where(iota_e == i1, -jnp.inf, lg)
    m2 = jnp.max(masked, axis=0, keepdims=True)
    i2 = jnp.min(jnp.where(masked == m2, iota_e, E), axis=0, keepdims=True)
    w_top = 1.0 / (1.0 + jnp.exp(m2 - m1))
    wgt_ref[...] = jnp.concatenate([w_top, 1.0 - w_top], axis=0)

    e_pair = jnp.concatenate([i1, i2], axis=1)
    iota_ep = lax.broadcasted_iota(jnp.int32, (E, P), 0)
    o_scr[...] = (e_pair == iota_ep).astype(jnp.float32)
    utri = (lax.broadcasted_iota(jnp.int32, (RB, RB), 0)
            < lax.broadcasted_iota(jnp.int32, (RB, RB), 1)).astype(jnp.float32)

    def step(i, csum):
        blk = o_scr[:, pl.ds(i * RB, RB)]
        cum_scr[:, pl.ds(i * RB, RB)] = (
            jnp.dot(blk, utri, preferred_element_type=jnp.float32) + csum)
        return csum + jnp.sum(blk, axis=1, keepdims=True)

    counts = lax.fori_loop(0, P // RB, step, jnp.zeros((E, 1), jnp.float32))
    caps = jnp.ceil(counts / B) * B
    mask_lt = (lax.broadcasted_iota(jnp.int32, (E, E), 1)
               < lax.broadcasted_iota(jnp.int32, (E, E), 0)).astype(jnp.float32)
    offs = jnp.sum(caps.T * mask_lt, axis=1, keepdims=True)
    ends = offs + caps
    onehot = o_scr[...]
    rank = jnp.sum(onehot * cum_scr[...], axis=0, keepdims=True)
    off_sel = jnp.sum(onehot * offs, axis=0, keepdims=True)
    dest_ref[...] = (rank + off_sel).astype(jnp.int32)

    tot = jnp.sum(caps, axis=0, keepdims=True)
    tot_ref[...] = tot.astype(jnp.int32)
    blk_start = jnp.minimum(
        lax.broadcasted_iota(jnp.int32, (32, 1), 0)
        .astype(jnp.float32) * float(B), tot - float(B))
    be = jnp.sum((ends.T <= blk_start).astype(jnp.float32), axis=1,
                 keepdims=True)
    be_ref[...] = be.astype(jnp.int32)


def _router(logits):
    return pl.pallas_call(
        _router_body,
        out_shape=(
            jax.ShapeDtypeStruct((K, T), jnp.float32),
            jax.ShapeDtypeStruct((1, P), jnp.int32),
            jax.ShapeDtypeStruct((32, 1), jnp.int32),
            jax.ShapeDtypeStruct((1, 1), jnp.int32),
        ),
        scratch_shapes=[
            pltpu.VMEM((E, P), jnp.float32),
            pltpu.VMEM((E, P), jnp.float32),
        ],
    )(logits)


_CHUNK_S = T // NW
_CHUNK_G = P // NW
_HG = _CHUNK_G // 2
H2 = H // 2


@functools.lru_cache(maxsize=None)
def _sc_kernels():
    mesh = plsc.VectorSubcoreMesh(core_axis_name="c", subcore_axis_name="s")

    @functools.partial(
        pl.kernel,
        out_type=jax.ShapeDtypeStruct((PPAD, H), jnp.float32),
        mesh=mesh,
        scratch_types=[
            pltpu.VMEM((_CHUNK_S,), jnp.int32),
            pltpu.VMEM((_CHUNK_S,), jnp.int32),
            pltpu.VMEM((_CHUNK_S, H), jnp.float32),
            pltpu.SemaphoreType.DMA,
        ],
    )
    def sc_scatter(x_hbm, d_hbm, xs_hbm, idx0_v, idx1_v, rows_v, sem):
        wid = lax.axis_index("s") * NC + lax.axis_index("c")
        base = wid * _CHUNK_S
        pltpu.sync_copy(d_hbm.at[pl.ds(base, _CHUNK_S)], idx0_v)
        pltpu.sync_copy(d_hbm.at[pl.ds(T + base, _CHUNK_S)], idx1_v)
        pltpu.sync_copy(x_hbm.at[pl.ds(base, _CHUNK_S)], rows_v)
        c0 = pltpu.async_copy(rows_v, xs_hbm.at[idx0_v], sem)
        c1 = pltpu.async_copy(rows_v, xs_hbm.at[idx1_v], sem)
        c0.wait()
        c1.wait()

    @functools.partial(
        pl.kernel,
        out_type=jax.ShapeDtypeStruct((P, H), jnp.float32),
        mesh=mesh,
        scratch_types=[
            pltpu.VMEM((_HG,), jnp.int32),
            pltpu.VMEM((_HG,), jnp.int32),
            pltpu.VMEM((_HG, H), jnp.float32),
            pltpu.VMEM((_HG, H), jnp.float32),
            pltpu.SemaphoreType.DMA,
            pltpu.SemaphoreType.DMA,
        ],
    )
    def sc_gather(ys_hbm, d_hbm, g_hbm, idx0_v, idx1_v, rows0_v, rows1_v,
                  sem0, sem1):
        wid = lax.axis_index("s") * NC + lax.axis_index("c")
        base = wid * _CHUNK_G
        pltpu.sync_copy(d_hbm.at[pl.ds(base, _HG)], idx0_v)
        pltpu.sync_copy(d_hbm.at[pl.ds(base + _HG, _HG)], idx1_v)
        c0 = pltpu.async_copy(ys_hbm.at[idx0_v], rows0_v, sem0)
        c1 = pltpu.async_copy(ys_hbm.at[idx1_v], rows1_v, sem1)
        c0.wait()
        w0 = pltpu.async_copy(rows0_v, g_hbm.at[pl.ds(base, _HG)], sem0)
        c1.wait()
        w1 = pltpu.async_copy(rows1_v, g_hbm.at[pl.ds(base + _HG, _HG)], sem1)
        w0.wait()
        w1.wait()

    return sc_scatter, sc_gather


_DN = (((1,), (1,)), ((), ()))


def _ffn_body(be_ref, tot_ref, xs_ref, w1_ref, b1_ref, w2_ref, b2_ref, y_ref):
    @pl.when(pl.program_id(0) * B < tot_ref[0])
    def _():
        x = xs_ref[...].astype(jnp.bfloat16)
        h = lax.dot_general(x, w1_ref[0].astype(jnp.bfloat16), _DN,
                            preferred_element_type=jnp.float32)
        h = h + b1_ref[0]
        gate = h[:, :I]
        up = h[:, I:]
        act = (gate * jax.nn.sigmoid(gate) * up).astype(jnp.bfloat16)
        y = lax.dot_general(act, w2_ref[0].astype(jnp.bfloat16), _DN,
                            preferred_element_type=jnp.float32)
        y_ref[...] = y + b2_ref[0]


def _ffn(be, tot, x_sorted, w1, b1, w2, b2):
    grid_spec = pltpu.PrefetchScalarGridSpec(
        num_scalar_prefetch=2,
        grid=(NB,),
        in_specs=[
            pl.BlockSpec((B, H), lambda b, be, tot: (b, 0)),
            pl.BlockSpec((1, I2, H), lambda b, be, tot: (be[b], 0, 0)),
            pl.BlockSpec((1, 1, I2), lambda b, be, tot: (be[b], 0, 0)),
            pl.BlockSpec((1, H, I), lambda b, be, tot: (be[b], 0, 0)),
            pl.BlockSpec((1, 1, H), lambda b, be, tot: (be[b], 0, 0)),
        ],
        out_specs=pl.BlockSpec((B, H), lambda b, be, tot: (b, 0)),
    )
    return pl.pallas_call(
        _ffn_body,
        grid_spec=grid_spec,
        out_shape=jax.ShapeDtypeStruct((PPAD, H), jnp.float32),
    )(be, tot, x_sorted, w1, b1, w2, b2)


def _combine_body(g0_ref, g1_ref, w_ref, o_ref):
    w = w_ref[...].T
    o_ref[...] = w[:, 0:1] * g0_ref[...] + w[:, 1:2] * g1_ref[...]


def _combine(g, wgt):
    nblk = 8
    rb = T // nblk
    return pl.pallas_call(
        _combine_body,
        grid=(nblk,),
        in_specs=[
            pl.BlockSpec((rb, H), lambda i: (i, 0)),
            pl.BlockSpec((rb, H), lambda i: (i + nblk, 0)),
            pl.BlockSpec((K, rb), lambda i: (0, i)),
        ],
        out_specs=pl.BlockSpec((rb, H), lambda i: (i, 0)),
        out_shape=jax.ShapeDtypeStruct((T, H), jnp.float32),
    )(g, g, wgt)


def kernel(x, Wg, bg, w1, b1, w2, b2):
    sc_scatter, sc_gather = _sc_kernels()
    logits = x @ Wg.T + bg
    wgt, dest_row, be_col, tot_col = _router(logits)
    dflat = dest_row.reshape(P)
    be = be_col.reshape(32)
    tot = tot_col.reshape(1)
    x_sorted = sc_scatter(x, dflat)
    y_sorted = _ffn(be, tot, x_sorted, w1, b1.reshape(E, 1, I2), w2,
                    b2.reshape(E, 1, H))
    g = sc_gather(y_sorted, dflat)
    return _combine(g, wgt)


# speedup vs baseline: 1.0097x; 1.0097x over previous
import functools

import jax
import jax.numpy as jnp
from jax import lax
from jax.experimental import pallas as pl
from jax.experimental.pallas import tpu as pltpu
from jax.experimental.pallas import tpu_sc as plsc

E = 8
K = 2
H = 768
I = 1536
I2 = 2 * I
T = 2048
P = 2 * T
B = 576
RB = 512
NB = 15
NBE = T // B
PPAD = NB * B
NC = 2
NS = 16
NW = NC * NS


def _router_body(lg_ref, wgt_ref, dest_ref, be_ref, tot_ref, o_scr, cum_scr):
    lg = lg_ref[...].T
    iota_e = lax.broadcasted_iota(jnp.int32, (E, T), 0)
    m1 = jnp.max(lg, axis=0, keepdims=True)
    i1 = jnp.min(jnp.where(lg == m1, iota_e, E), axis=0, keepdims=True)
    masked = jnp.where(iota_e == i1, -jnp.inf, lg)
    m2 = jnp.max(masked, axis=0, keepdims=True)
    i2 = jnp.min(jnp.where(masked == m2, iota_e, E), axis=0, keepdims=True)
    w_top = 1.0 / (1.0 + jnp.exp(m2 - m1))
    wgt_ref[...] = jnp.concatenate([w_top, 1.0 - w_top], axis=0)

    e_pair = jnp.concatenate([i1, i2], axis=1)
    iota_ep = lax.broadcasted_iota(jnp.int32, (E, P), 0)
    o_scr[...] = (e_pair == iota_ep).astype(jnp.float32)
    utri = (lax.broadcasted_iota(jnp.int32, (RB, RB), 0)
            < lax.broadcasted_iota(jnp.int32, (RB, RB), 1)).astype(jnp.float32)

    def step(i, csum):
        blk = o_scr[:, pl.ds(i * RB, RB)]
        cum_scr[:, pl.ds(i * RB, RB)] = (
            jnp.dot(blk, utri, preferred_element_type=jnp.float32) + csum)
        return csum + jnp.sum(blk, axis=1, keepdims=True)

    counts = lax.fori_loop(0, P // RB, step, jnp.zeros((E, 1), jnp.float32))
    caps = jnp.ceil(counts / B) * B
    mask_lt = (lax.broadcasted_iota(jnp.int32, (E, E), 1)
               < lax.broadcasted_iota(jnp.int32, (E, E), 0)).astype(jnp.float32)
    offs = jnp.sum(caps.T * mask_lt, axis=1, keepdims=True)
    ends = offs + caps
    onehot = o_scr[...]
    rank = jnp.sum(onehot * cum_scr[...], axis=0, keepdims=True)
    off_sel = jnp.sum(onehot * offs, axis=0, keepdims=True)
    dest_ref[...] = (rank + off_sel).astype(jnp.int32)

    tot = jnp.sum(caps, axis=0, keepdims=True)
    tot_ref[...] = tot.astype(jnp.int32)
    blk_start = jnp.minimum(
        lax.broadcasted_iota(jnp.int32, (32, 1), 0)
        .astype(jnp.float32) * float(B), tot - float(B))
    be = jnp.sum((ends.T <= blk_start).astype(jnp.float32), axis=1,
                 keepdims=True)
    be_ref[...] = be.astype(jnp.int32)


def _router(logits):
    return pl.pallas_call(
        _router_body,
        out_shape=(
            jax.ShapeDtypeStruct((K, T), jnp.float32),
            jax.ShapeDtypeStruct((1, P), jnp.int32),
            jax.ShapeDtypeStruct((32, 1), jnp.int32),
            jax.ShapeDtypeStruct((1, 1), jnp.int32),
        ),
        scratch_shapes=[
            pltpu.VMEM((E, P), jnp.float32),
            pltpu.VMEM((E, P), jnp.float32),
        ],
    )(logits)


_CHUNK_S = T // NW
_CHUNK_G = P // NW
H2 = H // 2


@functools.lru_cache(maxsize=None)
def _sc_kernels():
    mesh = plsc.VectorSubcoreMesh(core_axis_name="c", subcore_axis_name="s")

    @functools.partial(
        pl.kernel,
        out_type=jax.ShapeDtypeStruct((PPAD, H), jnp.float32),
        mesh=mesh,
        scratch_types=[
            pltpu.VMEM((_CHUNK_S,), jnp.int32),
            pltpu.VMEM((_CHUNK_S,), jnp.int32),
            pltpu.VMEM((_CHUNK_S, H), jnp.float32),
            pltpu.SemaphoreType.DMA,
        ],
    )
    def sc_scatter(x_hbm, d_hbm, xs_hbm, idx0_v, idx1_v, rows_v, sem):
        wid = lax.axis_index("s") * NC + lax.axis_index("c")
        base = wid * _CHUNK_S
        pltpu.sync_copy(d_hbm.at[pl.ds(base, _CHUNK_S)], idx0_v)
        pltpu.sync_copy(d_hbm.at[pl.ds(T + base, _CHUNK_S)], idx1_v)
        pltpu.sync_copy(x_hbm.at[pl.ds(base, _CHUNK_S)], rows_v)
        c0 = pltpu.async_copy(rows_v, xs_hbm.at[idx0_v], sem)
        c1 = pltpu.async_copy(rows_v, xs_hbm.at[idx1_v], sem)
        c0.wait()
        c1.wait()

    @functools.partial(
        pl.kernel,
        out_type=jax.ShapeDtypeStruct((P, H), jnp.float32),
        mesh=mesh,
        scratch_types=[
            pltpu.VMEM((_CHUNK_G,), jnp.int32),
            pltpu.VMEM((_CHUNK_G, H), jnp.float32),
            pltpu.SemaphoreType.DMA,
        ],
    )
    def sc_gather(ys_hbm, d_hbm, g_hbm, idx_v, rows_v, sem):
        wid = lax.axis_index("s") * NC + lax.axis_index("c")
        base = wid * _CHUNK_G
        pltpu.sync_copy(d_hbm.at[pl.ds(base, _CHUNK_G)], idx_v)
        pltpu.async_copy(ys_hbm.at[idx_v], rows_v, sem).wait()
        pltpu.sync_copy(rows_v, g_hbm.at[pl.ds(base, _CHUNK_G)])

    return sc_scatter, sc_gather


_DN = (((1,), (1,)), ((), ()))


def _ffn_body(be_ref, tot_ref, xs_ref, w1_ref, b1_ref, w2_ref, b2_ref, y_ref):
    @pl.when(pl.program_id(0) * B < tot_ref[0])
    def _():
        x = xs_ref[...].astype(jnp.bfloat16)
        h = lax.dot_general(x, w1_ref[0].astype(jnp.bfloat16), _DN,
                            preferred_element_type=jnp.float32)
        h = h + b1_ref[0]
        gate = h[:, :I]
        up = h[:, I:]
        act = (gate * jax.nn.sigmoid(gate) * up).astype(jnp.bfloat16)
        y = lax.dot_general(act, w2_ref[0].astype(jnp.bfloat16), _DN,
                            preferred_element_type=jnp.float32)
        y_ref[...] = y + b2_ref[0]


def _ffn(be, tot, x_sorted, w1, b1, w2, b2):
    grid_spec = pltpu.PrefetchScalarGridSpec(
        num_scalar_prefetch=2,
        grid=(NB,),
        in_specs=[
            pl.BlockSpec((B, H), lambda b, be, tot: (b, 0)),
            pl.BlockSpec((1, I2, H), lambda b, be, tot: (be[b], 0, 0)),
            pl.BlockSpec((1, 1, I2), lambda b, be, tot: (be[b], 0, 0)),
            pl.BlockSpec((1, H, I), lambda b, be, tot: (be[b], 0, 0)),
            pl.BlockSpec((1, 1, H), lambda b, be, tot: (be[b], 0, 0)),
        ],
        out_specs=pl.BlockSpec((B, H), lambda b, be, tot: (b, 0)),
    )
    return pl.pallas_call(
        _ffn_body,
        grid_spec=grid_spec,
        out_shape=jax.ShapeDtypeStruct((PPAD, H), jnp.float32),
    )(be, tot, x_sorted, w1, b1, w2, b2)


def _combine_body(g0_ref, g1_ref, w_ref, o_ref):
    w = w_ref[...].T
    o_ref[...] = w[:, 0:1] * g0_ref[...] + w[:, 1:2] * g1_ref[...]


def _combine(g, wgt):
    nblk = 8
    rb = T // nblk
    return pl.pallas_call(
        _combine_body,
        grid=(nblk,),
        in_specs=[
            pl.BlockSpec((rb, H), lambda i: (i, 0)),
            pl.BlockSpec((rb, H), lambda i: (i + nblk, 0)),
            pl.BlockSpec((K, rb), lambda i: (0, i)),
        ],
        out_specs=pl.BlockSpec((rb, H), lambda i: (i, 0)),
        out_shape=jax.ShapeDtypeStruct((T, H), jnp.float32),
    )(g, g, wgt)


def kernel(x, Wg, bg, w1, b1, w2, b2):
    sc_scatter, sc_gather = _sc_kernels()
    logits = x @ Wg.T + bg
    wgt, dest_row, be_col, tot_col = _router(logits)
    dflat = dest_row.reshape(P)
    be = be_col.reshape(32)
    tot = tot_col.reshape(1)
    x_sorted = sc_scatter(x, dflat)
    y_sorted = _ffn(be, tot, x_sorted, w1, b1.reshape(E, 1, I2), w2,
                    b2.reshape(E, 1, H))
    g = sc_gather(y_sorted, dflat)
    return _combine(g, wgt)
